# baseline (speedup 1.0000x reference)
import functools

import jax
import jax.numpy as jnp
from jax import lax
from jax.experimental import pallas as pl
from jax.experimental.pallas import tpu as pltpu
from jax.experimental.pallas import tpu_sc as plsc

LANES = 16


def _gather_body(fx_hbm, seg_hbm, out_hbm, fx_v, seg_v, out_v, *, n, epw):
    wid = lax.axis_index("s") * 2 + lax.axis_index("c")
    base = wid * epw
    pltpu.sync_copy(fx_hbm, fx_v)
    pltpu.sync_copy(seg_hbm.at[pl.ds(base, epw)], seg_v)

    def body(g, carry):
        for b in range(4):
            o = (g * 4 + b) * LANES
            idx = seg_v[pl.ds(o, LANES)]
            out_v[pl.ds(o, LANES)] = plsc.load_gather(fx_v, [idx])
        return carry

    lax.fori_loop(0, epw // (4 * LANES), body, 0)
    pltpu.sync_copy(out_v, out_hbm.at[pl.ds(base, epw)])


def _sc_gather(fx_flat, seg, n, e):
    epw = e // 32
    mesh = plsc.VectorSubcoreMesh(
        core_axis_name="c", subcore_axis_name="s", num_cores=2, num_subcores=16
    )
    kern = functools.partial(
        pl.kernel,
        out_type=jax.ShapeDtypeStruct((e,), jnp.float32),
        mesh=mesh,
        scratch_types=[
            pltpu.VMEM((n,), jnp.float32),
            pltpu.VMEM((epw,), jnp.int32),
            pltpu.VMEM((epw,), jnp.float32),
        ],
        compiler_params=pltpu.CompilerParams(needs_layout_passes=False),
    )(functools.partial(_gather_body, n=n, epw=epw))
    return kern(fx_flat, seg)


def _edge_body(nb_ref, fxr_ref, wct_ref, par_ref, z_ref, zs_ref, *, off_blk):
    tt = lax.dot_general(
        wct_ref[...], nb_ref[...],
        (((1,), (1,)), ((), ())),
        preferred_element_type=jnp.float32,
    )
    par = par_ref[...]
    blk = nb_ref.shape[0]
    off = (pl.program_id(0) + off_blk) * blk
    loff = pl.program_id(0) * blk
    fx = fxr_ref[pl.ds(off, blk)]
    h = jnp.tanh(tt + fx * par[:, 0:1] + par[:, 1:2])
    lp = jnp.sum(h * par[:, 2:3], axis=0) + par[0, 3]
    s = tt[12, :]
    logit = 1.0 / (1.0 + jnp.exp(-lp))
    z = jnp.exp(logit)
    z_ref[pl.ds(loff, blk)] = z
    zs_ref[pl.ds(loff, blk)] = z * s


def _tc_edges(neighbours, fx_row, wct, par, eh, blk, off_blk):
    grid = eh // blk
    e = fx_row.shape[0]
    return pl.pallas_call(
        functools.partial(_edge_body, off_blk=off_blk),
        grid=(grid,),
        in_specs=[
            pl.BlockSpec((blk, 128), lambda i: (i + off_blk, 0)),
            pl.BlockSpec((e,), lambda i: (0,)),
            pl.BlockSpec((16, 128), lambda i: (0, 0)),
            pl.BlockSpec((16, 128), lambda i: (0, 0)),
        ],
        out_specs=[
            pl.BlockSpec((eh,), lambda i: (0,)),
            pl.BlockSpec((eh,), lambda i: (0,)),
        ],
        out_shape=[
            jax.ShapeDtypeStruct((eh,), jnp.float32),
            jax.ShapeDtypeStruct((eh,), jnp.float32),
        ],
    )(neighbours, fx_row, wct, par)


def _scatter_body(
    z_hbm, zs_hbm, seg_hbm, zero_hbm, out_hbm,
    z_v, zs_v, seg_v, den_sh, num_sh, sem,
    *, rpt, nsl,
):
    cid = lax.axis_index("c")
    sid = lax.axis_index("s")

    @pl.when(sid == 0)
    def _():
        pltpu.sync_copy(zero_hbm, den_sh)
        pltpu.sync_copy(zero_hbm, num_sh)

    plsc.subcore_barrier()

    base = (sid * 2 + cid) * rpt
    cps = [
        pltpu.async_copy(z_hbm.at[pl.ds(base * 128, rpt * 128)], z_v, sem),
        pltpu.async_copy(zs_hbm.at[pl.ds(base * 128, rpt * 128)], zs_v, sem),
        pltpu.async_copy(seg_hbm.at[pl.ds(base, rpt)], seg_v, sem),
    ]
    for cp in cps:
        cp.wait()

    unroll = 8

    def body(g, carry):
        cps = []
        for b in range(unroll):
            c = g * unroll + b
            idx = seg_v.at[c]
            src_z = z_v.at[pl.ds(c * 128, 128)]
            src_zs = zs_v.at[pl.ds(c * 128, 128)]
            cps.append(pltpu.async_copy(src_z, den_sh.at[idx], sem, add=True))
            cps.append(pltpu.async_copy(src_zs, num_sh.at[idx], sem, add=True))
        for cp in cps:
            cp.wait()
        return carry

    lax.fori_loop(0, rpt // unroll, body, 0)
    plsc.subcore_barrier()

    nb = sid * nsl
    pltpu.sync_copy(den_sh.at[pl.ds(nb, nsl)], out_hbm.at[cid, 0, pl.ds(nb, nsl)])
    pltpu.sync_copy(num_sh.at[pl.ds(nb, nsl)], out_hbm.at[cid, 1, pl.ds(nb, nsl)])


def _sc_scatter(z1d, zs1d, seg2d, zeros, npad):
    rows = seg2d.shape[0]
    rpt = rows // 32
    nsl = npad // 16
    mesh = plsc.VectorSubcoreMesh(
        core_axis_name="c", subcore_axis_name="s", num_cores=2, num_subcores=16
    )
    kern = functools.partial(
        pl.kernel,
        out_type=jax.ShapeDtypeStruct((2, 2, npad), jnp.float32),
        mesh=mesh,
        scratch_types=[
            pltpu.VMEM((rpt * 128,), jnp.float32),
            pltpu.VMEM((rpt * 128,), jnp.float32),
            pltpu.VMEM((rpt, 128), jnp.int32),
            pltpu.VMEM_SHARED((npad,), jnp.float32),
            pltpu.VMEM_SHARED((npad,), jnp.float32),
            pltpu.SemaphoreType.DMA,
        ],
        compiler_params=pltpu.CompilerParams(needs_layout_passes=False),
    )(functools.partial(_scatter_body, rpt=rpt, nsl=nsl))
    return kern(z1d, zs1d, seg2d, zeros)


def _combine_body(*refs):
    part_refs = refs[:-3]
    fxp_ref, par_ref, out_ref = refs[-3:]
    den = jnp.zeros_like(fxp_ref[...])
    num = jnp.zeros_like(den)
    for pr in part_refs:
        p = pr[...]
        den = den + p[0, 0] + p[1, 0]
        num = num + p[0, 1] + p[1, 1]
    fx = fxp_ref[...]
    wg0 = par_ref[0, 4]
    bgc = par_ref[0, 5]
    ratio = jnp.where(den > 0.5, num / jnp.maximum(den, 0.5), 0.0)
    out_ref[...] = fx * wg0 + ratio + bgc


def _tc_combine(parts, fx_pad, par, npad):
    return pl.pallas_call(
        _combine_body,
        out_shape=jax.ShapeDtypeStruct((npad,), jnp.float32),
    )(*parts, fx_pad, par)


def kernel(f_x, neighbours, segment_ids, W1, b1, W2, b2, Wg, bg):
    n, e = f_x.shape[0], neighbours.shape[0]
    seg = segment_ids.astype(jnp.int32)
    fx_flat = f_x[:, 0]

    wct = jnp.pad(
        jnp.concatenate([W1[:, 1:], Wg[:, 1:]], axis=0), ((0, 3), (0, 0))
    )
    top = jnp.pad(jnp.stack([W1[:, 0], b1, W2[0, :]], axis=1), ((0, 4), (0, 0)))
    sc = jnp.pad(
        jnp.concatenate([b2, Wg[0, :1], bg])[None, :], ((0, 15), (0, 0))
    )
    par = jnp.pad(jnp.concatenate([top, sc], axis=1), ((0, 0), (0, 122)))

    blk = 6400
    nblk = e // blk
    third = nblk // 3
    splits = [nblk - 2 * third, third, third]
    ehpad = 32 * 128 * 8 * pl.cdiv(max(splits) * blk, 32 * 128 * 8)
    npad = 16 * LANES * pl.cdiv(n, 16 * LANES)
    fx_pad = jnp.concatenate([fx_flat, jnp.zeros((npad - n,), jnp.float32)])
    zeros = jnp.zeros((npad,), jnp.float32)
    rows = ehpad // 128

    fx_e = _sc_gather(fx_flat, seg, n, e)

    parts = []
    off = 0
    for nb_chunk in splits:
        ec = nb_chunk * blk
        zh, zsh = _tc_edges(neighbours, fx_e, wct, par, ec, blk, off)
        zpad = jnp.zeros((ehpad - ec,), jnp.float32)
        segh = jnp.concatenate(
            [seg[off * blk:off * blk + ec], zpad.astype(jnp.int32)]
        )
        zp = jnp.concatenate([zh, zpad])
        zsp = jnp.concatenate([zsh, zpad])
        parts.append(_sc_scatter(zp, zsp, segh.reshape(rows, 128), zeros, npad))
        off += nb_chunk

    out = _tc_combine(parts, fx_pad, par, npad)
    return out[:n][:, None]


# speedup vs baseline: 1.2795x; 1.2795x over previous
import functools

import jax
import jax.numpy as jnp
from jax import lax
from jax.experimental import pallas as pl
from jax.experimental.pallas import tpu as pltpu
from jax.experimental.pallas import tpu_sc as plsc

LANES = 16


def _gather_body(fx_hbm, seg_hbm, out_hbm, fx_v, seg_v, out_v, *, n, epw):
    wid = lax.axis_index("s") * 2 + lax.axis_index("c")
    base = wid * epw
    pltpu.sync_copy(fx_hbm, fx_v)
    pltpu.sync_copy(seg_hbm.at[pl.ds(base, epw)], seg_v)

    def body(g, carry):
        for b in range(4):
            o = (g * 4 + b) * LANES
            idx = seg_v[pl.ds(o, LANES)]
            out_v[pl.ds(o, LANES)] = plsc.load_gather(fx_v, [idx])
        return carry

    lax.fori_loop(0, epw // (4 * LANES), body, 0)
    pltpu.sync_copy(out_v, out_hbm.at[pl.ds(base, epw)])


def _sc_gather(fx_flat, seg, n, e):
    epw = e // 32
    mesh = plsc.VectorSubcoreMesh(
        core_axis_name="c", subcore_axis_name="s", num_cores=2, num_subcores=16
    )
    kern = functools.partial(
        pl.kernel,
        out_type=jax.ShapeDtypeStruct((e,), jnp.float32),
        mesh=mesh,
        scratch_types=[
            pltpu.VMEM((n,), jnp.float32),
            pltpu.VMEM((epw,), jnp.int32),
            pltpu.VMEM((epw,), jnp.float32),
        ],
        compiler_params=pltpu.CompilerParams(needs_layout_passes=False),
    )(functools.partial(_gather_body, n=n, epw=epw))
    return kern(fx_flat, seg)


def _edge_body(nb_ref, fxr_ref, wct_ref, par_ref, z_ref, zs_ref, *, off_blk):
    tt = lax.dot_general(
        wct_ref[...], nb_ref[...],
        (((1,), (1,)), ((), ())),
        preferred_element_type=jnp.float32,
    )
    par = par_ref[...]
    blk = nb_ref.shape[0]
    off = (pl.program_id(0) + off_blk) * blk
    loff = pl.program_id(0) * blk
    fx = fxr_ref[pl.ds(off, blk)]
    h = jnp.tanh(tt + fx * par[:, 0:1] + par[:, 1:2])
    lp = jnp.sum(h * par[:, 2:3], axis=0) + par[0, 3]
    s = tt[12, :]
    logit = 1.0 / (1.0 + jnp.exp(-lp))
    z = jnp.exp(logit)
    z_ref[pl.ds(loff, blk)] = z
    zs_ref[pl.ds(loff, blk)] = z * s


def _tc_edges(neighbours, fx_row, wct, par, eh, blk, off_blk):
    grid = eh // blk
    e = fx_row.shape[0]
    return pl.pallas_call(
        functools.partial(_edge_body, off_blk=off_blk),
        grid=(grid,),
        in_specs=[
            pl.BlockSpec((blk, 128), lambda i: (i + off_blk, 0)),
            pl.BlockSpec((e,), lambda i: (0,)),
            pl.BlockSpec((16, 128), lambda i: (0, 0)),
            pl.BlockSpec((16, 128), lambda i: (0, 0)),
        ],
        out_specs=[
            pl.BlockSpec((eh,), lambda i: (0,)),
            pl.BlockSpec((eh,), lambda i: (0,)),
        ],
        out_shape=[
            jax.ShapeDtypeStruct((eh,), jnp.float32),
            jax.ShapeDtypeStruct((eh,), jnp.float32),
        ],
    )(neighbours, fx_row, wct, par)


def _scatter_body(
    z_hbm, zs_hbm, seg_hbm, zero_hbm, out_hbm,
    z_v, zs_v, seg_v, den_sh, num_sh, sem,
    *, rpt, nsl,
):
    cid = lax.axis_index("c")
    sid = lax.axis_index("s")

    @pl.when(sid == 0)
    def _():
        pltpu.sync_copy(zero_hbm, den_sh)
        pltpu.sync_copy(zero_hbm, num_sh)

    plsc.subcore_barrier()

    base = (sid * 2 + cid) * rpt
    cps = [
        pltpu.async_copy(z_hbm.at[pl.ds(base * 128, rpt * 128)], z_v, sem),
        pltpu.async_copy(zs_hbm.at[pl.ds(base * 128, rpt * 128)], zs_v, sem),
        pltpu.async_copy(seg_hbm.at[pl.ds(base, rpt)], seg_v, sem),
    ]
    for cp in cps:
        cp.wait()

    unroll = 8

    def body(g, carry):
        cps = []
        for b in range(unroll):
            c = g * unroll + b
            idx = seg_v.at[c]
            src_z = z_v.at[pl.ds(c * 128, 128)]
            src_zs = zs_v.at[pl.ds(c * 128, 128)]
            cps.append(pltpu.async_copy(src_z, den_sh.at[idx], sem, add=True))
            cps.append(pltpu.async_copy(src_zs, num_sh.at[idx], sem, add=True))
        for cp in cps:
            cp.wait()
        return carry

    lax.fori_loop(0, rpt // unroll, body, 0)
    plsc.subcore_barrier()

    nb = sid * nsl
    pltpu.sync_copy(den_sh.at[pl.ds(nb, nsl)], out_hbm.at[cid, 0, pl.ds(nb, nsl)])
    pltpu.sync_copy(num_sh.at[pl.ds(nb, nsl)], out_hbm.at[cid, 1, pl.ds(nb, nsl)])


def _sc_scatter(z1d, zs1d, seg2d, zeros, npad):
    rows = seg2d.shape[0]
    rpt = rows // 32
    nsl = npad // 16
    mesh = plsc.VectorSubcoreMesh(
        core_axis_name="c", subcore_axis_name="s", num_cores=2, num_subcores=16
    )
    kern = functools.partial(
        pl.kernel,
        out_type=jax.ShapeDtypeStruct((2, 2, npad), jnp.float32),
        mesh=mesh,
        scratch_types=[
            pltpu.VMEM((rpt * 128,), jnp.float32),
            pltpu.VMEM((rpt * 128,), jnp.float32),
            pltpu.VMEM((rpt, 128), jnp.int32),
            pltpu.VMEM_SHARED((npad,), jnp.float32),
            pltpu.VMEM_SHARED((npad,), jnp.float32),
            pltpu.SemaphoreType.DMA,
        ],
        compiler_params=pltpu.CompilerParams(needs_layout_passes=False),
    )(functools.partial(_scatter_body, rpt=rpt, nsl=nsl))
    return kern(z1d, zs1d, seg2d, zeros)


def _combine_body(*refs):
    part_refs = refs[:-3]
    fxp_ref, par_ref, out_ref = refs[-3:]
    den = jnp.zeros_like(fxp_ref[...])
    num = jnp.zeros_like(den)
    for pr in part_refs:
        p = pr[...]
        den = den + p[0, 0] + p[1, 0]
        num = num + p[0, 1] + p[1, 1]
    fx = fxp_ref[...]
    wg0 = par_ref[0, 4]
    bgc = par_ref[0, 5]
    ratio = jnp.where(den > 0.5, num / jnp.maximum(den, 0.5), 0.0)
    out_ref[...] = fx * wg0 + ratio + bgc


def _tc_combine(parts, fx_pad, par, npad):
    return pl.pallas_call(
        _combine_body,
        out_shape=jax.ShapeDtypeStruct((npad,), jnp.float32),
    )(*parts, fx_pad, par)


def kernel(f_x, neighbours, segment_ids, W1, b1, W2, b2, Wg, bg):
    n, e = f_x.shape[0], neighbours.shape[0]
    seg = segment_ids.astype(jnp.int32)
    fx_flat = f_x[:, 0]

    wct = jnp.pad(
        jnp.concatenate([W1[:, 1:], Wg[:, 1:]], axis=0), ((0, 3), (0, 0))
    )
    top = jnp.pad(jnp.stack([W1[:, 0], b1, W2[0, :]], axis=1), ((0, 4), (0, 0)))
    sc = jnp.pad(
        jnp.concatenate([b2, Wg[0, :1], bg])[None, :], ((0, 15), (0, 0))
    )
    par = jnp.pad(jnp.concatenate([top, sc], axis=1), ((0, 0), (0, 122)))

    blk = 6400
    nblk = e // blk
    half = nblk // 2
    splits = [nblk - half, half]
    ehpad = 32 * 128 * 8 * pl.cdiv(max(splits) * blk, 32 * 128 * 8)
    npad = 16 * LANES * pl.cdiv(n, 16 * LANES)
    fx_pad = jnp.concatenate([fx_flat, jnp.zeros((npad - n,), jnp.float32)])
    zeros = jnp.zeros((npad,), jnp.float32)
    rows = ehpad // 128

    fx_e = _sc_gather(fx_flat, seg, n, e)

    parts = []
    off = 0
    for nb_chunk in splits:
        ec = nb_chunk * blk
        zh, zsh = _tc_edges(neighbours, fx_e, wct, par, ec, blk, off)
        zpad = jnp.zeros((ehpad - ec,), jnp.float32)
        segh = jnp.concatenate(
            [seg[off * blk:off * blk + ec], zpad.astype(jnp.int32)]
        )
        zp = jnp.concatenate([zh, zpad])
        zsp = jnp.concatenate([zsh, zpad])
        parts.append(_sc_scatter(zp, zsp, segh.reshape(rows, 128), zeros, npad))
        off += nb_chunk

    out = _tc_combine(parts, fx_pad, par, npad)
    return out[:n][:, None]
